# baseline (speedup 1.0000x reference)
import functools

import jax
import jax.numpy as jnp
import numpy as np
from jax import lax
from jax.experimental import pallas as pl
from jax.experimental.pallas import tpu as pltpu
from jax.experimental.pallas import tpu_sc as plsc

CUTOFF = 5.0
LOG2 = float(np.log(2.0))


def _ssp(x):
    return jnp.log1p(jnp.exp(x)) - LOG2


def _filter_body(rbf_ref, distT_ref, wf1_ref, bf1_ref, wf2_ref, bf2_ref, out_ref):
    rbf = rbf_ref[...].reshape(-1, rbf_ref.shape[2])
    t = jnp.dot(rbf, wf1_ref[...], preferred_element_type=jnp.float32)
    t = _ssp(t + bf1_ref[...])
    t = jnp.dot(t, wf2_ref[...], preferred_element_type=jnp.float32) + bf2_ref[...]
    d = distT_ref[0]
    cut = 0.5 * (jnp.cos(d * (np.pi / CUTOFF)) + 1.0)
    cut = jnp.where(d < CUTOFF, cut, 0.0)
    m_stripes = d.shape[1]
    for m in range(m_stripes):
        out_ref[pl.ds(m * 128, 128), :] = t[m * 128:(m + 1) * 128, :] * cut[:, m:m + 1]


def _hw1_body(h_ref, w1_ref, b1_ref, out_ref):
    out_ref[...] = (
        jnp.dot(h_ref[...], w1_ref[...], preferred_element_type=jnp.float32)
        + b1_ref[...]
    )


def _final_body(*refs):
    (h_ref, w2_ref, b2_ref, w3_ref, b3_ref, out_ref) = refs[-6:]
    parts_refs = refs[:-6]
    agg = parts_refs[0][0] + parts_refs[0][1]
    for pr in parts_refs[1:]:
        agg = agg + pr[0] + pr[1]
    t = jnp.dot(agg, w2_ref[...], preferred_element_type=jnp.float32) + b2_ref[...]
    t = _ssp(t)
    t = jnp.dot(t, w3_ref[...], preferred_element_type=jnp.float32) + b3_ref[...]
    out_ref[...] = h_ref[...] + t


def _make_sc_agg(N, E, H, K, seg_base, seg_edges):
    NC, NS = 2, 16
    NW = NC * NS
    assert seg_edges % NW == 0
    EPW = seg_edges // NW
    assert EPW % K == 0 and K % 8 == 0 and K <= 128
    NCHUNK = EPW // K
    assert NCHUNK >= 3 and NCHUNK % 2 == 1
    ZB = 16
    assert N % ZB == 0
    NBLK = N // ZB
    BPT, BREM = NBLK // NS, NBLK % NS
    mesh = plsc.VectorSubcoreMesh(
        core_axis_name="c", subcore_axis_name="s", num_cores=NC, num_subcores=NS
    )

    buf_types = [
        pltpu.VMEM((K,), jnp.int32),
        pltpu.VMEM((K,), jnp.int32),
        pltpu.VMEM((K,), jnp.int32),
        pltpu.VMEM((K, H), jnp.float32),
        pltpu.VMEM((K, H), jnp.float32),
        pltpu.SemaphoreType.DMA,
        pltpu.SemaphoreType.DMA,
        pltpu.SemaphoreType.DMA,
    ]

    @functools.partial(
        pl.kernel,
        mesh=mesh,
        out_type=jax.ShapeDtypeStruct((NC, N, H), jnp.float32),
        scratch_types=buf_types + buf_types + [
            pltpu.VMEM((ZB, H), jnp.float32),
            pltpu.VMEM_SHARED((N, H), jnp.float32),
        ],
    )
    def sc_agg(hw1_hbm, coef_hbm, ei_hbm, out_hbm, *scr):
        nb = len(buf_types)
        names = ("col", "row", "rowsc", "rows", "coef", "sin", "sg", "ss")
        buf = [dict(zip(names, scr[:nb])), dict(zip(names, scr[nb:2 * nb]))]
        zerov, agg_sh = scr[2 * nb], scr[2 * nb + 1]
        cid = lax.axis_index("c")
        sid = lax.axis_index("s")
        wid = sid * NC + cid
        start_blk = sid * BPT + jnp.minimum(sid, BREM)
        cnt_blk = BPT + jnp.where(sid < BREM, 1, 0)

        zvec = jnp.zeros((16,), jnp.float32)

        def zset(t, _):
            r = t // (H // 16)
            c = (t % (H // 16)) * 16
            zerov[r, pl.ds(c, 16)] = zvec
            return 0

        lax.fori_loop(0, ZB * (H // 16), zset, 0)

        def zcopy(z, _):
            pltpu.sync_copy(zerov, agg_sh.at[pl.ds((start_blk + z) * ZB, ZB)])
            return 0

        lax.fori_loop(0, cnt_blk, zcopy, 0)
        plsc.subcore_barrier()

        def fire_inputs(j, b):
            base = wid * EPW + j * K
            abase = seg_base + base
            pltpu.async_copy(ei_hbm.at[pl.ds(E + abase, K)], buf[b]["col"], buf[b]["sin"])
            pltpu.async_copy(ei_hbm.at[pl.ds(abase, K)], buf[b]["row"], buf[b]["sin"])
            pltpu.async_copy(coef_hbm.at[pl.ds(base, K)], buf[b]["coef"], buf[b]["sin"])

        def wait_inputs(j, b):
            base = wid * EPW + j * K
            abase = seg_base + base
            pltpu.make_async_copy(ei_hbm.at[pl.ds(E + abase, K)], buf[b]["col"], buf[b]["sin"]).wait()
            pltpu.make_async_copy(ei_hbm.at[pl.ds(abase, K)], buf[b]["row"], buf[b]["sin"]).wait()
            pltpu.make_async_copy(coef_hbm.at[pl.ds(base, K)], buf[b]["coef"], buf[b]["sin"]).wait()

        def issue_gather(b):
            pltpu.async_copy(hw1_hbm.at[buf[b]["col"]], buf[b]["rows"], buf[b]["sg"])

        def wait_gather(b):
            pltpu.make_async_copy(hw1_hbm.at[buf[b]["col"]], buf[b]["rows"], buf[b]["sg"]).wait()

        def mul(b):
            rows, coef = buf[b]["rows"], buf[b]["coef"]

            def mbody(rr, _):
                r = rr * 4
                for u in range(4):
                    for cc in range(H // 16):
                        sl = pl.ds(cc * 16, 16)
                        rows[r + u, sl] = rows[r + u, sl] * coef[r + u, sl]
                return 0

            lax.fori_loop(0, K // 4, mbody, 0)

        def snap_rowidx(b):
            for i in range(K // 16):
                sl = pl.ds(i * 16, 16)
                buf[b]["rowsc"][sl] = buf[b]["row"][sl]

        def issue_scatter(b):
            pltpu.async_copy(buf[b]["rows"], agg_sh.at[buf[b]["rowsc"]], buf[b]["ss"], add=True)

        def wait_scatter(b):
            pltpu.make_async_copy(buf[b]["rows"], agg_sh.at[buf[b]["rowsc"]], buf[b]["ss"]).wait()

        def step(j, b, first, fire_next, do_next):
            if do_next:
                wait_inputs(j + 1, 1 - b)
            if not first:
                wait_scatter(1 - b)
            if do_next:
                issue_gather(1 - b)
            wait_gather(b)
            mul(b)
            snap_rowidx(b)
            issue_scatter(b)
            if fire_next:
                fire_inputs(j + 2, b)

        fire_inputs(0, 0)
        wait_inputs(0, 0)
        issue_gather(0)
        fire_inputs(1, 1)
        step(0, 0, True, True, True)

        def pair(i, _):
            j = 1 + 2 * i
            step(j, 1, False, True, True)
            step(j + 1, 0, False, True, True)
            return 0

        lax.fori_loop(0, (NCHUNK - 3) // 2, pair, 0)
        step(NCHUNK - 2, 1, False, False, True)
        step(NCHUNK - 1, 0, False, False, False)
        wait_scatter(0)
        plsc.subcore_barrier()

        def wout(z, _):
            r0 = (start_blk + z) * ZB
            pltpu.sync_copy(agg_sh.at[pl.ds(r0, ZB)], out_hbm.at[cid, pl.ds(r0, ZB)])
            return 0

        lax.fori_loop(0, cnt_blk, wout, 0)

    return sc_agg


def kernel(h, edge_index, edge_dist, edge_rbf, Wf1, bf1, Wf2, bf2,
           W1, b1, W2, b2, W3, b3):
    N, H = h.shape
    E, NG = edge_rbf.shape
    ei = edge_index.astype(jnp.int32).reshape(2 * E)

    NSEG = 5 if E % (5 * 32 * 80) == 0 else 1
    ES = E // NSEG
    BE = next(b for b in (3200, 1280, 640, 128) if ES % b == 0)
    grid_e = E // BE
    seg_blocks = ES // BE
    distT = edge_dist.reshape(E // 128, 128).T.reshape(128, grid_e, BE // 128).transpose(1, 0, 2)
    rbf8 = edge_rbf.reshape(E // 8, 8, NG)
    bf1_2d = bf1.reshape(1, H)
    bf2_2d = bf2.reshape(1, H)
    b1_2d = b1.reshape(1, H)
    b2_2d = b2.reshape(1, H)
    b3_2d = b3.reshape(1, H)

    hw1 = pl.pallas_call(
        _hw1_body,
        out_shape=jax.ShapeDtypeStruct((N, H), jnp.float32),
    )(h, W1, b1_2d)

    parts_list = []
    for s in range(NSEG):
        off = s * seg_blocks
        coef_s = pl.pallas_call(
            _filter_body,
            grid=(seg_blocks,),
            in_specs=[
                pl.BlockSpec((BE // 8, 8, NG), lambda i, off=off: (off + i, 0, 0)),
                pl.BlockSpec((1, 128, BE // 128), lambda i, off=off: (off + i, 0, 0)),
                pl.BlockSpec((NG, H), lambda i: (0, 0)),
                pl.BlockSpec((1, H), lambda i: (0, 0)),
                pl.BlockSpec((H, H), lambda i: (0, 0)),
                pl.BlockSpec((1, H), lambda i: (0, 0)),
            ],
            out_specs=pl.BlockSpec((BE, H), lambda i: (i, 0)),
            out_shape=jax.ShapeDtypeStruct((ES, H), jnp.float32),
        )(rbf8, distT, Wf1, bf1_2d, Wf2, bf2_2d)
        parts_list.append(
            _make_sc_agg(N, E, H, K=80, seg_base=s * ES, seg_edges=ES)(hw1, coef_s, ei)
        )

    BN = next(b for b in (2000, 1000, 500, 128, 8) if N % b == 0)
    grid_n = N // BN
    out = pl.pallas_call(
        _final_body,
        grid=(grid_n,),
        in_specs=[pl.BlockSpec((2, BN, H), lambda i: (0, i, 0)) for _ in parts_list]
        + [
            pl.BlockSpec((BN, H), lambda i: (i, 0)),
            pl.BlockSpec((H, H), lambda i: (0, 0)),
            pl.BlockSpec((1, H), lambda i: (0, 0)),
            pl.BlockSpec((H, H), lambda i: (0, 0)),
            pl.BlockSpec((1, H), lambda i: (0, 0)),
        ],
        out_specs=pl.BlockSpec((BN, H), lambda i: (i, 0)),
        out_shape=jax.ShapeDtypeStruct((N, H), jnp.float32),
    )(*parts_list, h, W2, b2_2d, W3, b3_2d)
    return out


# speedup vs baseline: 1.2202x; 1.2202x over previous
import functools

import jax
import jax.numpy as jnp
import numpy as np
from jax import lax
from jax.experimental import pallas as pl
from jax.experimental.pallas import tpu as pltpu
from jax.experimental.pallas import tpu_sc as plsc

CUTOFF = 5.0
LOG2 = float(np.log(2.0))


def _ssp(x):
    return jnp.log1p(jnp.exp(x)) - LOG2


def _filter_body(rbf_ref, distT_ref, wf1_ref, bf1_ref, wf2_ref, bf2_ref, out_ref):
    rbf = rbf_ref[...].reshape(-1, rbf_ref.shape[2])
    t = jnp.dot(rbf, wf1_ref[...], preferred_element_type=jnp.float32)
    t = _ssp(t + bf1_ref[...])
    t = jnp.dot(t, wf2_ref[...], preferred_element_type=jnp.float32) + bf2_ref[...]
    d = distT_ref[0]
    cut = 0.5 * (jnp.cos(d * (np.pi / CUTOFF)) + 1.0)
    cut = jnp.where(d < CUTOFF, cut, 0.0)
    m_stripes = d.shape[1]
    for m in range(m_stripes):
        out_ref[pl.ds(m * 128, 128), :] = t[m * 128:(m + 1) * 128, :] * cut[:, m:m + 1]


def _hw1_body(h_ref, w1_ref, b1_ref, out_ref):
    out_ref[...] = (
        jnp.dot(h_ref[...], w1_ref[...], preferred_element_type=jnp.float32)
        + b1_ref[...]
    )


def _final_body(*refs):
    (h_ref, w2_ref, b2_ref, w3_ref, b3_ref, out_ref) = refs[-6:]
    parts_refs = refs[:-6]
    agg = parts_refs[0][0] + parts_refs[0][1]
    for pr in parts_refs[1:]:
        agg = agg + pr[0] + pr[1]
    t = jnp.dot(agg, w2_ref[...], preferred_element_type=jnp.float32) + b2_ref[...]
    t = _ssp(t)
    t = jnp.dot(t, w3_ref[...], preferred_element_type=jnp.float32) + b3_ref[...]
    out_ref[...] = h_ref[...] + t


def _make_sc_agg(N, E, H, K, seg_base, seg_edges):
    NC, NS = 2, 16
    NW = NC * NS
    assert seg_edges % NW == 0
    EPW = seg_edges // NW
    assert EPW % K == 0 and K % 8 == 0 and K <= 128
    NCHUNK = EPW // K
    assert NCHUNK >= 3 and NCHUNK % 2 == 1
    ZB = 16
    assert N % ZB == 0
    NBLK = N // ZB
    BPT, BREM = NBLK // NS, NBLK % NS
    mesh = plsc.VectorSubcoreMesh(
        core_axis_name="c", subcore_axis_name="s", num_cores=NC, num_subcores=NS
    )

    buf_types = [
        pltpu.VMEM((K,), jnp.int32),
        pltpu.VMEM((K,), jnp.int32),
        pltpu.VMEM((K,), jnp.int32),
        pltpu.VMEM((K, H), jnp.float32),
        pltpu.VMEM((K, H), jnp.float32),
        pltpu.SemaphoreType.DMA,
        pltpu.SemaphoreType.DMA,
        pltpu.SemaphoreType.DMA,
    ]

    @functools.partial(
        pl.kernel,
        mesh=mesh,
        out_type=jax.ShapeDtypeStruct((NC, N, H), jnp.float32),
        scratch_types=buf_types + buf_types + [
            pltpu.VMEM((ZB, H), jnp.float32),
            pltpu.VMEM_SHARED((N, H), jnp.float32),
            pltpu.SemaphoreType.DMA,
        ],
    )
    def sc_agg(hw1_hbm, coef_hbm, ei_hbm, out_hbm, *scr):
        nb = len(buf_types)
        names = ("col", "row", "rowsc", "rows", "coef", "sin", "sg", "ss")
        buf = [dict(zip(names, scr[:nb])), dict(zip(names, scr[nb:2 * nb]))]
        zerov, agg_sh, zw_sem = scr[2 * nb], scr[2 * nb + 1], scr[2 * nb + 2]
        cid = lax.axis_index("c")
        sid = lax.axis_index("s")
        wid = sid * NC + cid
        start_blk = sid * BPT + jnp.minimum(sid, BREM)
        cnt_blk = BPT + jnp.where(sid < BREM, 1, 0)

        zvec = jnp.zeros((16,), jnp.float32)

        def zset(t, _):
            r = t // (H // 16)
            c = (t % (H // 16)) * 16
            zerov[r, pl.ds(c, 16)] = zvec
            return 0

        lax.fori_loop(0, ZB * (H // 16), zset, 0)

        def zfire(z, _):
            pltpu.async_copy(zerov, agg_sh.at[pl.ds((start_blk + z) * ZB, ZB)], zw_sem)
            return 0

        def zwait(z, _):
            pltpu.make_async_copy(zerov, agg_sh.at[pl.ds((start_blk + z) * ZB, ZB)], zw_sem).wait()
            return 0

        lax.fori_loop(0, cnt_blk, zfire, 0)
        lax.fori_loop(0, cnt_blk, zwait, 0)
        plsc.subcore_barrier()

        def fire_inputs(j, b):
            base = wid * EPW + j * K
            abase = seg_base + base
            pltpu.async_copy(ei_hbm.at[pl.ds(E + abase, K)], buf[b]["col"], buf[b]["sin"])
            pltpu.async_copy(ei_hbm.at[pl.ds(abase, K)], buf[b]["row"], buf[b]["sin"])
            pltpu.async_copy(coef_hbm.at[pl.ds(base, K)], buf[b]["coef"], buf[b]["sin"])

        def wait_inputs(j, b):
            base = wid * EPW + j * K
            abase = seg_base + base
            pltpu.make_async_copy(ei_hbm.at[pl.ds(E + abase, K)], buf[b]["col"], buf[b]["sin"]).wait()
            pltpu.make_async_copy(ei_hbm.at[pl.ds(abase, K)], buf[b]["row"], buf[b]["sin"]).wait()
            pltpu.make_async_copy(coef_hbm.at[pl.ds(base, K)], buf[b]["coef"], buf[b]["sin"]).wait()

        def issue_gather(b):
            pltpu.async_copy(hw1_hbm.at[buf[b]["col"]], buf[b]["rows"], buf[b]["sg"])

        def wait_gather(b):
            pltpu.make_async_copy(hw1_hbm.at[buf[b]["col"]], buf[b]["rows"], buf[b]["sg"]).wait()

        def mul(b):
            rows, coef = buf[b]["rows"], buf[b]["coef"]

            def mbody(rr, _):
                r = rr * 4
                for u in range(4):
                    for cc in range(H // 16):
                        sl = pl.ds(cc * 16, 16)
                        rows[r + u, sl] = rows[r + u, sl] * coef[r + u, sl]
                return 0

            lax.fori_loop(0, K // 4, mbody, 0)

        def snap_rowidx(b):
            for i in range(K // 16):
                sl = pl.ds(i * 16, 16)
                buf[b]["rowsc"][sl] = buf[b]["row"][sl]

        def issue_scatter(b):
            pltpu.async_copy(buf[b]["rows"], agg_sh.at[buf[b]["rowsc"]], buf[b]["ss"], add=True)

        def wait_scatter(b):
            pltpu.make_async_copy(buf[b]["rows"], agg_sh.at[buf[b]["rowsc"]], buf[b]["ss"]).wait()

        def step(j, b, first, fire_next, do_next):
            if do_next:
                wait_inputs(j + 1, 1 - b)
            if not first:
                wait_scatter(1 - b)
            if do_next:
                issue_gather(1 - b)
            wait_gather(b)
            mul(b)
            snap_rowidx(b)
            issue_scatter(b)
            if fire_next:
                fire_inputs(j + 2, b)

        fire_inputs(0, 0)
        wait_inputs(0, 0)
        issue_gather(0)
        fire_inputs(1, 1)
        step(0, 0, True, True, True)

        def pair(i, _):
            j = 1 + 2 * i
            step(j, 1, False, True, True)
            step(j + 1, 0, False, True, True)
            return 0

        lax.fori_loop(0, (NCHUNK - 3) // 2, pair, 0)
        step(NCHUNK - 2, 1, False, False, True)
        step(NCHUNK - 1, 0, False, False, False)
        wait_scatter(0)
        plsc.subcore_barrier()

        def wfire(z, _):
            r0 = (start_blk + z) * ZB
            pltpu.async_copy(agg_sh.at[pl.ds(r0, ZB)], out_hbm.at[cid, pl.ds(r0, ZB)], zw_sem)
            return 0

        def wwait(z, _):
            r0 = (start_blk + z) * ZB
            pltpu.make_async_copy(agg_sh.at[pl.ds(r0, ZB)], out_hbm.at[cid, pl.ds(r0, ZB)], zw_sem).wait()
            return 0

        lax.fori_loop(0, cnt_blk, wfire, 0)
        lax.fori_loop(0, cnt_blk, wwait, 0)

    return sc_agg


def kernel(h, edge_index, edge_dist, edge_rbf, Wf1, bf1, Wf2, bf2,
           W1, b1, W2, b2, W3, b3):
    N, H = h.shape
    E, NG = edge_rbf.shape
    ei = edge_index.astype(jnp.int32).reshape(2 * E)

    NSEG = 5 if E % (5 * 32 * 80) == 0 else 1
    ES = E // NSEG
    BE = next(b for b in (3200, 1280, 640, 128) if ES % b == 0)
    grid_e = E // BE
    seg_blocks = ES // BE
    distT = edge_dist.reshape(E // 128, 128).T.reshape(128, grid_e, BE // 128).transpose(1, 0, 2)
    rbf8 = edge_rbf.reshape(E // 8, 8, NG)
    bf1_2d = bf1.reshape(1, H)
    bf2_2d = bf2.reshape(1, H)
    b1_2d = b1.reshape(1, H)
    b2_2d = b2.reshape(1, H)
    b3_2d = b3.reshape(1, H)

    hw1 = pl.pallas_call(
        _hw1_body,
        out_shape=jax.ShapeDtypeStruct((N, H), jnp.float32),
    )(h, W1, b1_2d)

    parts_list = []
    for s in range(NSEG):
        off = s * seg_blocks
        coef_s = pl.pallas_call(
            _filter_body,
            grid=(seg_blocks,),
            in_specs=[
                pl.BlockSpec((BE // 8, 8, NG), lambda i, off=off: (off + i, 0, 0)),
                pl.BlockSpec((1, 128, BE // 128), lambda i, off=off: (off + i, 0, 0)),
                pl.BlockSpec((NG, H), lambda i: (0, 0)),
                pl.BlockSpec((1, H), lambda i: (0, 0)),
                pl.BlockSpec((H, H), lambda i: (0, 0)),
                pl.BlockSpec((1, H), lambda i: (0, 0)),
            ],
            out_specs=pl.BlockSpec((BE, H), lambda i: (i, 0)),
            out_shape=jax.ShapeDtypeStruct((ES, H), jnp.float32),
        )(rbf8, distT, Wf1, bf1_2d, Wf2, bf2_2d)
        parts_list.append(
            _make_sc_agg(N, E, H, K=80, seg_base=s * ES, seg_edges=ES)(hw1, coef_s, ei)
        )

    BN = next(b for b in (2000, 1000, 500, 128, 8) if N % b == 0)
    grid_n = N // BN
    out = pl.pallas_call(
        _final_body,
        grid=(grid_n,),
        in_specs=[pl.BlockSpec((2, BN, H), lambda i: (0, i, 0)) for _ in parts_list]
        + [
            pl.BlockSpec((BN, H), lambda i: (i, 0)),
            pl.BlockSpec((H, H), lambda i: (0, 0)),
            pl.BlockSpec((1, H), lambda i: (0, 0)),
            pl.BlockSpec((H, H), lambda i: (0, 0)),
            pl.BlockSpec((1, H), lambda i: (0, 0)),
        ],
        out_specs=pl.BlockSpec((BN, H), lambda i: (i, 0)),
        out_shape=jax.ShapeDtypeStruct((N, H), jnp.float32),
    )(*parts_list, h, W2, b2_2d, W3, b3_2d)
    return out
